# baseline (speedup 1.0000x reference)
import functools

import jax
import jax.numpy as jnp
from jax import lax
from jax.experimental import pallas as pl
from jax.experimental.pallas import tpu as pltpu
from jax.experimental.pallas import tpu_sc as plsc

N = 10000
E = 160000
COLS = 128
NB_ROWS = 1000
K_EDGE = 50
NCHUNK = 100
NCHUNK_PAD = 104
CNT_K = 125
CNT_CHUNKS = 40
ROWS_PER_TEC = 624
ZCHUNK = 208
TAIL_BASE = 16 * ROWS_PER_TEC


def _pre_body(x_ref, pos_ref, w1h_ref, w1p_ref, b1_ref, a_ref, q_ref):
    q = (pos_ref[:, 0:1] * w1p_ref[0, 0:1, :]
         + pos_ref[:, 1:2] * w1p_ref[0, 1:2, :])
    a = jnp.dot(x_ref[...].astype(jnp.bfloat16),
                w1h_ref[0].astype(jnp.bfloat16),
                preferred_element_type=jnp.float32)
    a_ref[0] = a + q + b1_ref[0, 0][None, :]
    q_ref[0] = q


def _pre_call(x, pos, w1h, w1p, b1, nb):
    fin = x.shape[1]
    w1h_t = w1h.reshape(fin, nb, COLS).transpose(1, 0, 2)
    w1p_t = w1p.reshape(2, nb, COLS).transpose(1, 0, 2)
    b1_t = b1.reshape(nb, 1, COLS)
    grid = (N // NB_ROWS, nb)
    out_shape = [jax.ShapeDtypeStruct((nb, N, COLS), jnp.float32),
                 jax.ShapeDtypeStruct((nb, N, COLS), jnp.float32)]
    return pl.pallas_call(
        _pre_body,
        grid=grid,
        in_specs=[
            pl.BlockSpec((NB_ROWS, fin), lambda i, j: (i, 0)),
            pl.BlockSpec((NB_ROWS, 2), lambda i, j: (i, 0)),
            pl.BlockSpec((1, fin, COLS), lambda i, j: (j, 0, 0)),
            pl.BlockSpec((1, 2, COLS), lambda i, j: (j, 0, 0)),
            pl.BlockSpec((1, 1, COLS), lambda i, j: (j, 0, 0)),
        ],
        out_specs=[
            pl.BlockSpec((1, NB_ROWS, COLS), lambda i, j: (j, i, 0)),
            pl.BlockSpec((1, NB_ROWS, COLS), lambda i, j: (j, i, 0)),
        ],
        out_shape=out_shape,
    )(x, pos, w1h_t, w1p_t, b1_t)


def _post_body(nb, do_relu, s_ref, cnt_ref, w2_ref, b2_ref, y_ref):
    cnt = jnp.sum(cnt_ref[...], axis=(0, 2)) * (1.0 / COLS)
    inv = 1.0 / jnp.maximum(cnt, 1.0)
    bmask = jnp.where(cnt > 0.0, 1.0, 0.0)
    acc = jnp.zeros((NB_ROWS, w2_ref.shape[1]), jnp.float32)
    for b in range(nb):
        m = (s_ref[0, b] + s_ref[1, b]) * inv[:, None]
        acc = acc + jnp.dot(m.astype(jnp.bfloat16),
                            w2_ref[b * COLS:(b + 1) * COLS, :]
                            .astype(jnp.bfloat16),
                            preferred_element_type=jnp.float32)
    y = acc + b2_ref[...][None, :] * bmask[:, None]
    if do_relu:
        y = jnp.maximum(y, 0.0)
    y_ref[...] = y


def _post_call(s_part, cnt_part, w2, b2, nb, do_relu):
    fout = w2.shape[1]
    return pl.pallas_call(
        functools.partial(_post_body, nb, do_relu),
        grid=(N // NB_ROWS,),
        in_specs=[
            pl.BlockSpec((2, nb, NB_ROWS, COLS), lambda i: (0, 0, i, 0)),
            pl.BlockSpec((2, NB_ROWS, COLS), lambda i: (0, i, 0)),
            pl.BlockSpec((nb * COLS, fout), lambda i: (0, 0)),
            pl.BlockSpec((fout,), lambda i: (0,)),
        ],
        out_specs=pl.BlockSpec((NB_ROWS, fout), lambda i: (i, 0)),
        out_shape=jax.ShapeDtypeStruct((N, fout), jnp.float32),
    )(s_part, cnt_part, w2, b2)


def _mlp_body(x_ref, w1_ref, b1_ref, w2_ref, b2_ref, w3_ref, b3_ref, o_ref):
    bf = jnp.bfloat16
    y = jnp.dot(x_ref[...].astype(bf), w1_ref[...].astype(bf),
                preferred_element_type=jnp.float32)
    y = jnp.maximum(y + b1_ref[...][None, :], 0.0)
    y = jnp.dot(y.astype(bf), w2_ref[...].astype(bf),
                preferred_element_type=jnp.float32)
    y = jnp.maximum(y + b2_ref[...][None, :], 0.0)
    y = jnp.dot(y.astype(bf), w3_ref[...].astype(bf),
                preferred_element_type=jnp.float32)
    o_ref[...] = y + b3_ref[...][None, :]


def _mlp_call(x, w1, b1, w2, b2, w3, b3):
    return pl.pallas_call(
        _mlp_body,
        grid=(N // NB_ROWS,),
        in_specs=[
            pl.BlockSpec((NB_ROWS, 1024), lambda i: (i, 0)),
            pl.BlockSpec((1024, 1024), lambda i: (0, 0)),
            pl.BlockSpec((1024,), lambda i: (0,)),
            pl.BlockSpec((1024, 512), lambda i: (0, 0)),
            pl.BlockSpec((512,), lambda i: (0,)),
            pl.BlockSpec((512, 4), lambda i: (0, 0)),
            pl.BlockSpec((4,), lambda i: (0,)),
        ],
        out_specs=pl.BlockSpec((NB_ROWS, 4), lambda i: (i, 0)),
        out_shape=jax.ShapeDtypeStruct((N, 4), jnp.float32),
    )(x, w1, b1, w2, b2, w3, b3)


def _fill_const(ref, rows, cols, val):
    @pl.loop(0, rows)
    def _(j):
        for l in range(cols // 16):
            ref[j, pl.ds(l * 16, 16)] = jnp.full((16,), val, jnp.float32)


def _tiled_copy(s, src_fn, dst_fn):
    for z in range(ROWS_PER_TEC // ZCHUNK):
        base = pl.multiple_of(s * ROWS_PER_TEC + z * ZCHUNK, 16)
        pltpu.sync_copy(src_fn(base, z, ZCHUNK), dst_fn(base, z, ZCHUNK))

    @pl.when(s == 15)
    def _():
        pltpu.sync_copy(src_fn(TAIL_BASE, 3, 16), dst_fn(TAIL_BASE, 3, 16))


def _zoff(z):
    return ZCHUNK * z if z < 3 else ROWS_PER_TEC


def _edge_body(nb, a_ref, q_ref, idx_ref, z_ref, zd_ref, spart_ref,
               i0, i1, i2, i3, a0, q0, m0, a1, q1, m1,
               gi0, gi1, g0, g1, s0, s1, acc):
    c = lax.axis_index("core")
    s = lax.axis_index("subcore")
    g0_row = (c * 16 + s) * NCHUNK

    islot = (i0, i1, i2, i3)
    abuf = (a0, a1)
    qbuf = (q0, q1)
    mbuf = (m0, m1)
    isem = (gi0, gi1)
    gsem = (g0, g1)
    ssem = (s0, s1)

    for b in range(nb):
        _tiled_copy(s, lambda o, z, r: z_ref.at[pl.ds(_zoff(z), r)],
                    lambda o, z, r: acc.at[pl.ds(o, r)])
        plsc.subcore_barrier()

        tab_a = a_ref.at[b]
        tab_q = q_ref.at[b]

        for p in range(2):
            pltpu.sync_copy(idx_ref.at[g0_row + p], islot[p])
            pltpu.async_copy(tab_a.at[islot[p].at[0]], abuf[p], gsem[p])
            pltpu.async_copy(tab_q.at[islot[p].at[1]], qbuf[p], gsem[p])

        @pl.loop(0, NCHUNK, step=4)
        def _(c0):
            for u in range(4):
                p = u % 2
                inow = islot[u]
                inxt = islot[(u + 2) % 4]
                ch = c0 + u
                pltpu.make_async_copy(
                    tab_a.at[inow.at[0]], abuf[p], gsem[p]).wait()
                pltpu.make_async_copy(
                    tab_q.at[inow.at[1]], qbuf[p], gsem[p]).wait()

                @pl.when(ch >= 2)
                def _():
                    pltpu.make_async_copy(
                        zd_ref.at[0], mbuf[p], ssem[p]).wait()

                @pl.when(ch + 2 < NCHUNK)
                def _():
                    pltpu.async_copy(idx_ref.at[g0_row + ch + 2],
                                     inxt, isem[p])

                @pl.loop(0, K_EDGE)
                def _(j):
                    for l in range(COLS // 16):
                        sl = pl.ds(l * 16, 16)
                        mbuf[p][j, sl] = jnp.maximum(
                            abuf[p][j, sl] - qbuf[p][j, sl], 0.0)

                pltpu.async_copy(mbuf[p], acc.at[inow.at[1]], ssem[p],
                                 add=True)

                @pl.when(ch + 2 < NCHUNK)
                def _():
                    pltpu.make_async_copy(idx_ref.at[g0_row + ch + 2],
                                          inxt, isem[p]).wait()
                    pltpu.async_copy(tab_a.at[inxt.at[0]],
                                     abuf[p], gsem[p])
                    pltpu.async_copy(tab_q.at[inxt.at[1]],
                                     qbuf[p], gsem[p])

        for p in range(2):
            pltpu.make_async_copy(zd_ref.at[0], mbuf[p], ssem[p]).wait()

        plsc.subcore_barrier()
        _tiled_copy(s, lambda o, z, r: acc.at[pl.ds(o, r)],
                    lambda o, z, r: spart_ref.at[c, b, pl.ds(o, r)])
        plsc.subcore_barrier()


def _edge_call(a_t, q_t, idxc, nb):
    mesh = plsc.VectorSubcoreMesh(core_axis_name="core",
                                  subcore_axis_name="subcore")
    fn = pl.kernel(
        functools.partial(_edge_body, nb),
        mesh=mesh,
        out_type=[jax.ShapeDtypeStruct((2, nb, N, COLS), jnp.float32)],
        scratch_types=[
            pltpu.VMEM((2, K_EDGE), jnp.int32),
            pltpu.VMEM((2, K_EDGE), jnp.int32),
            pltpu.VMEM((2, K_EDGE), jnp.int32),
            pltpu.VMEM((2, K_EDGE), jnp.int32),
            pltpu.VMEM((K_EDGE, COLS), jnp.float32),
            pltpu.VMEM((K_EDGE, COLS), jnp.float32),
            pltpu.VMEM((K_EDGE, COLS), jnp.float32),
            pltpu.VMEM((K_EDGE, COLS), jnp.float32),
            pltpu.VMEM((K_EDGE, COLS), jnp.float32),
            pltpu.VMEM((K_EDGE, COLS), jnp.float32),
            pltpu.SemaphoreType.DMA,
            pltpu.SemaphoreType.DMA,
            pltpu.SemaphoreType.DMA,
            pltpu.SemaphoreType.DMA,
            pltpu.SemaphoreType.DMA,
            pltpu.SemaphoreType.DMA,
            pltpu.VMEM_SHARED((N, COLS), jnp.float32),
        ],
    )
    z = jnp.zeros((ROWS_PER_TEC + 16, COLS), jnp.float32)
    zd = jnp.zeros((2, K_EDGE, COLS), jnp.float32)
    return fn(a_t, q_t, idxc, z, zd)[0]


def _cnt_body(dst_ref, z_ref, cnt_ref, dst_v, ones_b, cacc):
    c = lax.axis_index("core")
    s = lax.axis_index("subcore")
    row0 = c * 640 + s * CNT_CHUNKS
    pltpu.sync_copy(dst_ref.at[pl.ds(row0, CNT_CHUNKS)], dst_v)
    _fill_const(ones_b, CNT_K, COLS, 1.0)
    _tiled_copy(s, lambda o, z, r: z_ref.at[pl.ds(_zoff(z), r)],
                lambda o, z, r: cacc.at[pl.ds(o, r)])
    plsc.subcore_barrier()

    @pl.loop(0, CNT_CHUNKS)
    def _(chunk):
        pltpu.sync_copy(ones_b, cacc.at[dst_v.at[chunk]], add=True)

    plsc.subcore_barrier()
    _tiled_copy(s, lambda o, z, r: cacc.at[pl.ds(o, r)],
                lambda o, z, r: cnt_ref.at[c, pl.ds(o, r)])


def _cnt_call(dst2d):
    mesh = plsc.VectorSubcoreMesh(core_axis_name="core",
                                  subcore_axis_name="subcore")
    fn = pl.kernel(
        _cnt_body,
        mesh=mesh,
        out_type=[jax.ShapeDtypeStruct((2, N, COLS), jnp.float32)],
        scratch_types=[
            pltpu.VMEM((CNT_CHUNKS, CNT_K), jnp.int32),
            pltpu.VMEM((CNT_K, COLS), jnp.float32),
            pltpu.VMEM_SHARED((N, COLS), jnp.float32),
        ],
    )
    z = jnp.zeros((ROWS_PER_TEC + 16, COLS), jnp.float32)
    return fn(dst2d, z)[0]


def _layer(x, pos, idxc, w1, b1, w2, b2, cnt_part, do_relu):
    fin = x.shape[1]
    hid = w1.shape[1]
    nb = hid // COLS
    a_t, q_t = _pre_call(x, pos, w1[:fin], w1[fin:], b1, nb)
    s_part = _edge_call(a_t, q_t, idxc, nb)
    return _post_call(s_part, cnt_part, w2, b2, nb, do_relu)


def kernel(h, pos, edge_index,
           c1_w1, c1_b1, c1_w2, c1_b2,
           c2_w1, c2_b1, c2_w2, c2_b2,
           c3_w1, c3_b1, c3_w2, c3_b2,
           c4_w1, c4_b1, c4_w2, c4_b2,
           m_w1, m_b1, m_w2, m_b2, m_w3, m_b3):
    ei = edge_index.astype(jnp.int32)
    idxc = jnp.stack([ei[0].reshape(E // K_EDGE, K_EDGE),
                      ei[1].reshape(E // K_EDGE, K_EDGE)], axis=1)
    dst2d_cnt = ei[1].reshape(E // CNT_K, CNT_K)

    cnt_part = _cnt_call(dst2d_cnt)
    x = _layer(h, pos, idxc, c1_w1, c1_b1, c1_w2, c1_b2, cnt_part, True)
    x = _layer(x, pos, idxc, c2_w1, c2_b1, c2_w2, c2_b2, cnt_part, True)
    x = _layer(x, pos, idxc, c3_w1, c3_b1, c3_w2, c3_b2, cnt_part, True)
    x = _layer(x, pos, idxc, c4_w1, c4_b1, c4_w2, c4_b2, cnt_part, True)
    return _mlp_call(x, m_w1, m_b1, m_w2, m_b2, m_w3, m_b3)


# speedup vs baseline: 1.0054x; 1.0054x over previous
import functools

import jax
import jax.numpy as jnp
from jax import lax
from jax.experimental import pallas as pl
from jax.experimental.pallas import tpu as pltpu
from jax.experimental.pallas import tpu_sc as plsc

N = 10000
E = 160000
COLS = 128
NB_ROWS = 1000
K_EDGE = 50
NCHUNK = 100
NCHUNK_PAD = 104
CNT_K = 125
CNT_CHUNKS = 40
ROWS_PER_TEC = 624
ZCHUNK = 208
TAIL_BASE = 16 * ROWS_PER_TEC


def _pre_body(x_ref, pos_ref, w1h_ref, w1p_ref, b1_ref, a_ref, q_ref):
    q = (pos_ref[:, 0:1] * w1p_ref[0, 0:1, :]
         + pos_ref[:, 1:2] * w1p_ref[0, 1:2, :])
    a = jnp.dot(x_ref[...], w1h_ref[0],
                preferred_element_type=jnp.float32)
    a_ref[0] = a + q + b1_ref[0, 0][None, :]
    q_ref[0] = q


def _pre_call(x, pos, w1h, w1p, b1, nb):
    fin = x.shape[1]
    w1h_t = w1h.reshape(fin, nb, COLS).transpose(1, 0, 2)
    w1p_t = w1p.reshape(2, nb, COLS).transpose(1, 0, 2)
    b1_t = b1.reshape(nb, 1, COLS)
    grid = (N // NB_ROWS, nb)
    out_shape = [jax.ShapeDtypeStruct((nb, N, COLS), jnp.float32),
                 jax.ShapeDtypeStruct((nb, N, COLS), jnp.float32)]
    return pl.pallas_call(
        _pre_body,
        grid=grid,
        in_specs=[
            pl.BlockSpec((NB_ROWS, fin), lambda i, j: (i, 0)),
            pl.BlockSpec((NB_ROWS, 2), lambda i, j: (i, 0)),
            pl.BlockSpec((1, fin, COLS), lambda i, j: (j, 0, 0)),
            pl.BlockSpec((1, 2, COLS), lambda i, j: (j, 0, 0)),
            pl.BlockSpec((1, 1, COLS), lambda i, j: (j, 0, 0)),
        ],
        out_specs=[
            pl.BlockSpec((1, NB_ROWS, COLS), lambda i, j: (j, i, 0)),
            pl.BlockSpec((1, NB_ROWS, COLS), lambda i, j: (j, i, 0)),
        ],
        out_shape=out_shape,
    )(x, pos, w1h_t, w1p_t, b1_t)


def _post_body(nb, do_relu, s_ref, cnt_ref, w2_ref, b2_ref, y_ref):
    cnt = jnp.sum(cnt_ref[...], axis=(0, 2)) * (1.0 / COLS)
    inv = 1.0 / jnp.maximum(cnt, 1.0)
    bmask = jnp.where(cnt > 0.0, 1.0, 0.0)
    acc = jnp.zeros((NB_ROWS, w2_ref.shape[1]), jnp.float32)
    for b in range(nb):
        m = (s_ref[0, b] + s_ref[1, b]) * inv[:, None]
        acc = acc + jnp.dot(m, w2_ref[b * COLS:(b + 1) * COLS, :],
                            preferred_element_type=jnp.float32)
    y = acc + b2_ref[...][None, :] * bmask[:, None]
    if do_relu:
        y = jnp.maximum(y, 0.0)
    y_ref[...] = y


def _post_call(s_part, cnt_part, w2, b2, nb, do_relu):
    fout = w2.shape[1]
    return pl.pallas_call(
        functools.partial(_post_body, nb, do_relu),
        grid=(N // NB_ROWS,),
        in_specs=[
            pl.BlockSpec((2, nb, NB_ROWS, COLS), lambda i: (0, 0, i, 0)),
            pl.BlockSpec((2, NB_ROWS, COLS), lambda i: (0, i, 0)),
            pl.BlockSpec((nb * COLS, fout), lambda i: (0, 0)),
            pl.BlockSpec((fout,), lambda i: (0,)),
        ],
        out_specs=pl.BlockSpec((NB_ROWS, fout), lambda i: (i, 0)),
        out_shape=jax.ShapeDtypeStruct((N, fout), jnp.float32),
    )(s_part, cnt_part, w2, b2)


def _mlp_body(x_ref, w1_ref, b1_ref, w2_ref, b2_ref, w3_ref, b3_ref, o_ref):
    y = jnp.dot(x_ref[...], w1_ref[...], preferred_element_type=jnp.float32)
    y = jnp.maximum(y + b1_ref[...][None, :], 0.0)
    y = jnp.dot(y, w2_ref[...], preferred_element_type=jnp.float32)
    y = jnp.maximum(y + b2_ref[...][None, :], 0.0)
    y = jnp.dot(y, w3_ref[...], preferred_element_type=jnp.float32)
    o_ref[...] = y + b3_ref[...][None, :]


def _mlp_call(x, w1, b1, w2, b2, w3, b3):
    return pl.pallas_call(
        _mlp_body,
        grid=(N // NB_ROWS,),
        in_specs=[
            pl.BlockSpec((NB_ROWS, 1024), lambda i: (i, 0)),
            pl.BlockSpec((1024, 1024), lambda i: (0, 0)),
            pl.BlockSpec((1024,), lambda i: (0,)),
            pl.BlockSpec((1024, 512), lambda i: (0, 0)),
            pl.BlockSpec((512,), lambda i: (0,)),
            pl.BlockSpec((512, 4), lambda i: (0, 0)),
            pl.BlockSpec((4,), lambda i: (0,)),
        ],
        out_specs=pl.BlockSpec((NB_ROWS, 4), lambda i: (i, 0)),
        out_shape=jax.ShapeDtypeStruct((N, 4), jnp.float32),
    )(x, w1, b1, w2, b2, w3, b3)


def _fill_const(ref, rows, cols, val):
    @pl.loop(0, rows)
    def _(j):
        for l in range(cols // 16):
            ref[j, pl.ds(l * 16, 16)] = jnp.full((16,), val, jnp.float32)


def _tiled_copy(s, src_fn, dst_fn):
    for z in range(ROWS_PER_TEC // ZCHUNK):
        base = pl.multiple_of(s * ROWS_PER_TEC + z * ZCHUNK, 16)
        pltpu.sync_copy(src_fn(base, z, ZCHUNK), dst_fn(base, z, ZCHUNK))

    @pl.when(s == 15)
    def _():
        pltpu.sync_copy(src_fn(TAIL_BASE, 3, 16), dst_fn(TAIL_BASE, 3, 16))


def _zoff(z):
    return ZCHUNK * z if z < 3 else ROWS_PER_TEC


def _tiled_copy_async(s, src_fn, dst_fn, sem):
    for issue in (True, False):
        for z in range(ROWS_PER_TEC // ZCHUNK):
            base = pl.multiple_of(s * ROWS_PER_TEC + z * ZCHUNK, 16)
            cp = pltpu.make_async_copy(src_fn(base, z, ZCHUNK),
                                       dst_fn(base, z, ZCHUNK), sem)
            cp.start() if issue else cp.wait()

        @pl.when(s == 15)
        def _():
            cp = pltpu.make_async_copy(src_fn(TAIL_BASE, 3, 16),
                                       dst_fn(TAIL_BASE, 3, 16), sem)
            cp.start() if issue else cp.wait()


def _edge_body(nb, a_ref, q_ref, idx_ref, z_ref, zd_ref, spart_ref,
               i0, i1, i2, i3, a0, q0, m0, a1, q1, m1,
               gi0, gi1, g0, g1, s0, s1, acc):
    c = lax.axis_index("core")
    s = lax.axis_index("subcore")
    g0_row = (c * 16 + s) * NCHUNK

    islot = (i0, i1, i2, i3)
    abuf = (a0, a1)
    qbuf = (q0, q1)
    mbuf = (m0, m1)
    isem = (gi0, gi1)
    gsem = (g0, g1)
    ssem = (s0, s1)

    for b in range(nb):
        _tiled_copy_async(s, lambda o, z, r: z_ref.at[pl.ds(_zoff(z), r)],
                          lambda o, z, r: acc.at[pl.ds(o, r)], gsem[0])
        plsc.subcore_barrier()

        tab_a = a_ref.at[b]
        tab_q = q_ref.at[b]

        for p in range(2):
            pltpu.sync_copy(idx_ref.at[g0_row + p], islot[p])
            pltpu.async_copy(tab_a.at[islot[p].at[0]], abuf[p], gsem[p])
            pltpu.async_copy(tab_q.at[islot[p].at[1]], qbuf[p], gsem[p])

        @pl.loop(0, NCHUNK, step=4)
        def _(c0):
            for u in range(4):
                p = u % 2
                inow = islot[u]
                inxt = islot[(u + 2) % 4]
                ch = c0 + u
                pltpu.make_async_copy(
                    tab_a.at[inow.at[0]], abuf[p], gsem[p]).wait()
                pltpu.make_async_copy(
                    tab_q.at[inow.at[1]], qbuf[p], gsem[p]).wait()

                @pl.when(ch >= 2)
                def _():
                    pltpu.make_async_copy(
                        zd_ref.at[0], mbuf[p], ssem[p]).wait()

                @pl.when(ch + 2 < NCHUNK)
                def _():
                    pltpu.async_copy(idx_ref.at[g0_row + ch + 2],
                                     inxt, isem[p])

                @pl.loop(0, K_EDGE)
                def _(j):
                    for l in range(COLS // 16):
                        sl = pl.ds(l * 16, 16)
                        mbuf[p][j, sl] = jnp.maximum(
                            abuf[p][j, sl] - qbuf[p][j, sl], 0.0)

                pltpu.async_copy(mbuf[p], acc.at[inow.at[1]], ssem[p],
                                 add=True)

                @pl.when(ch + 2 < NCHUNK)
                def _():
                    pltpu.make_async_copy(idx_ref.at[g0_row + ch + 2],
                                          inxt, isem[p]).wait()
                    pltpu.async_copy(tab_a.at[inxt.at[0]],
                                     abuf[p], gsem[p])
                    pltpu.async_copy(tab_q.at[inxt.at[1]],
                                     qbuf[p], gsem[p])

        for p in range(2):
            pltpu.make_async_copy(zd_ref.at[0], mbuf[p], ssem[p]).wait()

        plsc.subcore_barrier()
        _tiled_copy_async(s, lambda o, z, r: acc.at[pl.ds(o, r)],
                          lambda o, z, r: spart_ref.at[c, b, pl.ds(o, r)],
                          gsem[0])
        plsc.subcore_barrier()


def _edge_call(a_t, q_t, idxc, nb):
    mesh = plsc.VectorSubcoreMesh(core_axis_name="core",
                                  subcore_axis_name="subcore")
    fn = pl.kernel(
        functools.partial(_edge_body, nb),
        mesh=mesh,
        out_type=[jax.ShapeDtypeStruct((2, nb, N, COLS), jnp.float32)],
        scratch_types=[
            pltpu.VMEM((2, K_EDGE), jnp.int32),
            pltpu.VMEM((2, K_EDGE), jnp.int32),
            pltpu.VMEM((2, K_EDGE), jnp.int32),
            pltpu.VMEM((2, K_EDGE), jnp.int32),
            pltpu.VMEM((K_EDGE, COLS), jnp.float32),
            pltpu.VMEM((K_EDGE, COLS), jnp.float32),
            pltpu.VMEM((K_EDGE, COLS), jnp.float32),
            pltpu.VMEM((K_EDGE, COLS), jnp.float32),
            pltpu.VMEM((K_EDGE, COLS), jnp.float32),
            pltpu.VMEM((K_EDGE, COLS), jnp.float32),
            pltpu.SemaphoreType.DMA,
            pltpu.SemaphoreType.DMA,
            pltpu.SemaphoreType.DMA,
            pltpu.SemaphoreType.DMA,
            pltpu.SemaphoreType.DMA,
            pltpu.SemaphoreType.DMA,
            pltpu.VMEM_SHARED((N, COLS), jnp.float32),
        ],
    )
    z = jnp.zeros((ROWS_PER_TEC + 16, COLS), jnp.float32)
    zd = jnp.zeros((2, K_EDGE, COLS), jnp.float32)
    return fn(a_t, q_t, idxc, z, zd)[0]


def _cnt_body(dst_ref, z_ref, cnt_ref, dst_v, ones_b, cacc):
    c = lax.axis_index("core")
    s = lax.axis_index("subcore")
    row0 = c * 640 + s * CNT_CHUNKS
    pltpu.sync_copy(dst_ref.at[pl.ds(row0, CNT_CHUNKS)], dst_v)
    _fill_const(ones_b, CNT_K, COLS, 1.0)
    _tiled_copy(s, lambda o, z, r: z_ref.at[pl.ds(_zoff(z), r)],
                lambda o, z, r: cacc.at[pl.ds(o, r)])
    plsc.subcore_barrier()

    @pl.loop(0, CNT_CHUNKS)
    def _(chunk):
        pltpu.sync_copy(ones_b, cacc.at[dst_v.at[chunk]], add=True)

    plsc.subcore_barrier()
    _tiled_copy(s, lambda o, z, r: cacc.at[pl.ds(o, r)],
                lambda o, z, r: cnt_ref.at[c, pl.ds(o, r)])


def _cnt_call(dst2d):
    mesh = plsc.VectorSubcoreMesh(core_axis_name="core",
                                  subcore_axis_name="subcore")
    fn = pl.kernel(
        _cnt_body,
        mesh=mesh,
        out_type=[jax.ShapeDtypeStruct((2, N, COLS), jnp.float32)],
        scratch_types=[
            pltpu.VMEM((CNT_CHUNKS, CNT_K), jnp.int32),
            pltpu.VMEM((CNT_K, COLS), jnp.float32),
            pltpu.VMEM_SHARED((N, COLS), jnp.float32),
        ],
    )
    z = jnp.zeros((ROWS_PER_TEC + 16, COLS), jnp.float32)
    return fn(dst2d, z)[0]


def _layer(x, pos, idxc, w1, b1, w2, b2, cnt_part, do_relu):
    fin = x.shape[1]
    hid = w1.shape[1]
    nb = hid // COLS
    a_t, q_t = _pre_call(x, pos, w1[:fin], w1[fin:], b1, nb)
    s_part = _edge_call(a_t, q_t, idxc, nb)
    return _post_call(s_part, cnt_part, w2, b2, nb, do_relu)


def kernel(h, pos, edge_index,
           c1_w1, c1_b1, c1_w2, c1_b2,
           c2_w1, c2_b1, c2_w2, c2_b2,
           c3_w1, c3_b1, c3_w2, c3_b2,
           c4_w1, c4_b1, c4_w2, c4_b2,
           m_w1, m_b1, m_w2, m_b2, m_w3, m_b3):
    ei = edge_index.astype(jnp.int32)
    idxc = jnp.stack([ei[0].reshape(E // K_EDGE, K_EDGE),
                      ei[1].reshape(E // K_EDGE, K_EDGE)], axis=1)
    dst2d_cnt = ei[1].reshape(E // CNT_K, CNT_K)

    cnt_part = _cnt_call(dst2d_cnt)
    x = _layer(h, pos, idxc, c1_w1, c1_b1, c1_w2, c1_b2, cnt_part, True)
    x = _layer(x, pos, idxc, c2_w1, c2_b1, c2_w2, c2_b2, cnt_part, True)
    x = _layer(x, pos, idxc, c3_w1, c3_b1, c3_w2, c3_b2, cnt_part, True)
    x = _layer(x, pos, idxc, c4_w1, c4_b1, c4_w2, c4_b2, cnt_part, True)
    return _mlp_call(x, m_w1, m_b1, m_w2, m_b2, m_w3, m_b3)


# speedup vs baseline: 1.0204x; 1.0149x over previous
import functools

import jax
import jax.numpy as jnp
from jax import lax
from jax.experimental import pallas as pl
from jax.experimental.pallas import tpu as pltpu
from jax.experimental.pallas import tpu_sc as plsc

N = 10000
E = 160000
COLS = 128
NB_ROWS = 1000
K_EDGE = 50
NCHUNK = 100
NCHUNK_PAD = 104
CNT_K = 125
CNT_CHUNKS = 40
ROWS_PER_TEC = 624
ZCHUNK = 208
TAIL_BASE = 16 * ROWS_PER_TEC


def _pre_body(x_ref, pos_ref, w1h_ref, w1p_ref, b1_ref, a_ref, q_ref):
    q = (pos_ref[:, 0:1] * w1p_ref[0, 0:1, :]
         + pos_ref[:, 1:2] * w1p_ref[0, 1:2, :])
    a = jnp.dot(x_ref[...], w1h_ref[0],
                preferred_element_type=jnp.float32)
    a_ref[0] = a + q + b1_ref[0, 0][None, :]
    q_ref[0] = q


def _pre_call(x, pos, w1h, w1p, b1, nb):
    fin = x.shape[1]
    w1h_t = w1h.reshape(fin, nb, COLS).transpose(1, 0, 2)
    w1p_t = w1p.reshape(2, nb, COLS).transpose(1, 0, 2)
    b1_t = b1.reshape(nb, 1, COLS)
    grid = (N // NB_ROWS, nb)
    out_shape = [jax.ShapeDtypeStruct((nb, N, COLS), jnp.float32),
                 jax.ShapeDtypeStruct((nb, N, COLS), jnp.float32)]
    return pl.pallas_call(
        _pre_body,
        grid=grid,
        in_specs=[
            pl.BlockSpec((NB_ROWS, fin), lambda i, j: (i, 0)),
            pl.BlockSpec((NB_ROWS, 2), lambda i, j: (i, 0)),
            pl.BlockSpec((1, fin, COLS), lambda i, j: (j, 0, 0)),
            pl.BlockSpec((1, 2, COLS), lambda i, j: (j, 0, 0)),
            pl.BlockSpec((1, 1, COLS), lambda i, j: (j, 0, 0)),
        ],
        out_specs=[
            pl.BlockSpec((1, NB_ROWS, COLS), lambda i, j: (j, i, 0)),
            pl.BlockSpec((1, NB_ROWS, COLS), lambda i, j: (j, i, 0)),
        ],
        out_shape=out_shape,
    )(x, pos, w1h_t, w1p_t, b1_t)


def _post_body(nb, do_relu, s_ref, cnt_ref, w2_ref, b2_ref, y_ref):
    cnt = jnp.sum(cnt_ref[...], axis=(0, 2)) * (1.0 / COLS)
    inv = 1.0 / jnp.maximum(cnt, 1.0)
    bmask = jnp.where(cnt > 0.0, 1.0, 0.0)
    acc = jnp.zeros((NB_ROWS, w2_ref.shape[1]), jnp.float32)
    for b in range(nb):
        m = (s_ref[0, b] + s_ref[1, b]) * inv[:, None]
        acc = acc + jnp.dot(m, w2_ref[b * COLS:(b + 1) * COLS, :],
                            preferred_element_type=jnp.float32)
    y = acc + b2_ref[...][None, :] * bmask[:, None]
    if do_relu:
        y = jnp.maximum(y, 0.0)
    y_ref[...] = y


def _post_call(s_part, cnt_part, w2, b2, nb, do_relu):
    fout = w2.shape[1]
    return pl.pallas_call(
        functools.partial(_post_body, nb, do_relu),
        grid=(N // NB_ROWS,),
        in_specs=[
            pl.BlockSpec((2, nb, NB_ROWS, COLS), lambda i: (0, 0, i, 0)),
            pl.BlockSpec((2, NB_ROWS, COLS), lambda i: (0, i, 0)),
            pl.BlockSpec((nb * COLS, fout), lambda i: (0, 0)),
            pl.BlockSpec((fout,), lambda i: (0,)),
        ],
        out_specs=pl.BlockSpec((NB_ROWS, fout), lambda i: (i, 0)),
        out_shape=jax.ShapeDtypeStruct((N, fout), jnp.float32),
    )(s_part, cnt_part, w2, b2)


def _mlp_body(x_ref, w1_ref, b1_ref, w2_ref, b2_ref, w3_ref, b3_ref, o_ref):
    y = jnp.dot(x_ref[...], w1_ref[...], preferred_element_type=jnp.float32)
    y = jnp.maximum(y + b1_ref[...][None, :], 0.0)
    y = jnp.dot(y, w2_ref[...], preferred_element_type=jnp.float32)
    y = jnp.maximum(y + b2_ref[...][None, :], 0.0)
    y = jnp.dot(y, w3_ref[...], preferred_element_type=jnp.float32)
    o_ref[...] = y + b3_ref[...][None, :]


def _mlp_call(x, w1, b1, w2, b2, w3, b3):
    return pl.pallas_call(
        _mlp_body,
        grid=(N // NB_ROWS,),
        in_specs=[
            pl.BlockSpec((NB_ROWS, 1024), lambda i: (i, 0)),
            pl.BlockSpec((1024, 1024), lambda i: (0, 0)),
            pl.BlockSpec((1024,), lambda i: (0,)),
            pl.BlockSpec((1024, 512), lambda i: (0, 0)),
            pl.BlockSpec((512,), lambda i: (0,)),
            pl.BlockSpec((512, 4), lambda i: (0, 0)),
            pl.BlockSpec((4,), lambda i: (0,)),
        ],
        out_specs=pl.BlockSpec((NB_ROWS, 4), lambda i: (i, 0)),
        out_shape=jax.ShapeDtypeStruct((N, 4), jnp.float32),
    )(x, w1, b1, w2, b2, w3, b3)


def _fill_const(ref, rows, cols, val):
    @pl.loop(0, rows)
    def _(j):
        for l in range(cols // 16):
            ref[j, pl.ds(l * 16, 16)] = jnp.full((16,), val, jnp.float32)


def _tiled_copy(s, src_fn, dst_fn):
    for z in range(ROWS_PER_TEC // ZCHUNK):
        base = pl.multiple_of(s * ROWS_PER_TEC + z * ZCHUNK, 16)
        pltpu.sync_copy(src_fn(base, z, ZCHUNK), dst_fn(base, z, ZCHUNK))

    @pl.when(s == 15)
    def _():
        pltpu.sync_copy(src_fn(TAIL_BASE, 3, 16), dst_fn(TAIL_BASE, 3, 16))


def _zoff(z):
    return ZCHUNK * z if z < 3 else ROWS_PER_TEC


def _tiled_copy_async(s, src_fn, dst_fn, sem):
    for issue in (True, False):
        for z in range(ROWS_PER_TEC // ZCHUNK):
            base = pl.multiple_of(s * ROWS_PER_TEC + z * ZCHUNK, 16)
            cp = pltpu.make_async_copy(src_fn(base, z, ZCHUNK),
                                       dst_fn(base, z, ZCHUNK), sem)
            cp.start() if issue else cp.wait()

        @pl.when(s == 15)
        def _():
            cp = pltpu.make_async_copy(src_fn(TAIL_BASE, 3, 16),
                                       dst_fn(TAIL_BASE, 3, 16), sem)
            cp.start() if issue else cp.wait()


def _edge_body(nb, a_ref, q_ref, idx_ref, z_ref, zd_ref, spart_ref,
               i0, i1, i2, i3, a0, q0, m0, a1, q1, m1,
               gi0, gi1, g0, g1, s0, s1, acc):
    c = lax.axis_index("core")
    s = lax.axis_index("subcore")
    g0_row = (c * 16 + s) * NCHUNK

    islot = (i0, i1, i2, i3)
    abuf = (a0, a1)
    qbuf = (q0, q1)
    mbuf = (m0, m1)
    isem = (gi0, gi1)
    gsem = (g0, g1)
    ssem = (s0, s1)

    for b in range(nb):
        _tiled_copy_async(s, lambda o, z, r: z_ref.at[pl.ds(_zoff(z), r)],
                          lambda o, z, r: acc.at[pl.ds(o, r)], gsem[0])
        plsc.subcore_barrier()

        tab_a = a_ref.at[b]
        tab_q = q_ref.at[b]

        for p in range(2):
            pltpu.sync_copy(idx_ref.at[g0_row + p], islot[p])
            pltpu.async_copy(tab_a.at[islot[p].at[0]], abuf[p], gsem[p])
            pltpu.async_copy(tab_q.at[islot[p].at[1]], qbuf[p], gsem[p])

        @pl.loop(0, NCHUNK, step=4)
        def _(c0):
            for u in range(4):
                p = u % 2
                inow = islot[u]
                inxt = islot[(u + 2) % 4]
                ch = c0 + u
                pltpu.make_async_copy(
                    tab_a.at[inow.at[0]], abuf[p], gsem[p]).wait()
                pltpu.make_async_copy(
                    tab_q.at[inow.at[1]], qbuf[p], gsem[p]).wait()

                @pl.when(ch >= 2)
                def _():
                    pltpu.make_async_copy(
                        zd_ref.at[0], mbuf[p], ssem[p]).wait()

                @pl.when(ch + 2 < NCHUNK)
                def _():
                    pltpu.async_copy(idx_ref.at[g0_row + ch + 2],
                                     inxt, isem[p])

                @pl.loop(0, K_EDGE, step=2)
                def _(j0):
                    for dj in range(2):
                        j = j0 + dj
                        for l in range(COLS // 16):
                            sl = pl.ds(l * 16, 16)
                            mbuf[p][j, sl] = jnp.maximum(
                                abuf[p][j, sl] - qbuf[p][j, sl], 0.0)

                pltpu.async_copy(mbuf[p], acc.at[inow.at[1]], ssem[p],
                                 add=True)

                @pl.when(ch + 2 < NCHUNK)
                def _():
                    pltpu.make_async_copy(idx_ref.at[g0_row + ch + 2],
                                          inxt, isem[p]).wait()
                    pltpu.async_copy(tab_a.at[inxt.at[0]],
                                     abuf[p], gsem[p])
                    pltpu.async_copy(tab_q.at[inxt.at[1]],
                                     qbuf[p], gsem[p])

        for p in range(2):
            pltpu.make_async_copy(zd_ref.at[0], mbuf[p], ssem[p]).wait()

        plsc.subcore_barrier()
        _tiled_copy_async(s, lambda o, z, r: acc.at[pl.ds(o, r)],
                          lambda o, z, r: spart_ref.at[c, b, pl.ds(o, r)],
                          gsem[0])


def _edge_call(a_t, q_t, idxc, nb):
    mesh = plsc.VectorSubcoreMesh(core_axis_name="core",
                                  subcore_axis_name="subcore")
    fn = pl.kernel(
        functools.partial(_edge_body, nb),
        mesh=mesh,
        out_type=[jax.ShapeDtypeStruct((2, nb, N, COLS), jnp.float32)],
        scratch_types=[
            pltpu.VMEM((2, K_EDGE), jnp.int32),
            pltpu.VMEM((2, K_EDGE), jnp.int32),
            pltpu.VMEM((2, K_EDGE), jnp.int32),
            pltpu.VMEM((2, K_EDGE), jnp.int32),
            pltpu.VMEM((K_EDGE, COLS), jnp.float32),
            pltpu.VMEM((K_EDGE, COLS), jnp.float32),
            pltpu.VMEM((K_EDGE, COLS), jnp.float32),
            pltpu.VMEM((K_EDGE, COLS), jnp.float32),
            pltpu.VMEM((K_EDGE, COLS), jnp.float32),
            pltpu.VMEM((K_EDGE, COLS), jnp.float32),
            pltpu.SemaphoreType.DMA,
            pltpu.SemaphoreType.DMA,
            pltpu.SemaphoreType.DMA,
            pltpu.SemaphoreType.DMA,
            pltpu.SemaphoreType.DMA,
            pltpu.SemaphoreType.DMA,
            pltpu.VMEM_SHARED((N, COLS), jnp.float32),
        ],
    )
    z = jnp.zeros((ROWS_PER_TEC + 16, COLS), jnp.float32)
    zd = jnp.zeros((2, K_EDGE, COLS), jnp.float32)
    return fn(a_t, q_t, idxc, z, zd)[0]


def _cnt_body(dst_ref, z_ref, cnt_ref, dst_v, ones_b, cacc):
    c = lax.axis_index("core")
    s = lax.axis_index("subcore")
    row0 = c * 640 + s * CNT_CHUNKS
    pltpu.sync_copy(dst_ref.at[pl.ds(row0, CNT_CHUNKS)], dst_v)
    _fill_const(ones_b, CNT_K, COLS, 1.0)
    _tiled_copy(s, lambda o, z, r: z_ref.at[pl.ds(_zoff(z), r)],
                lambda o, z, r: cacc.at[pl.ds(o, r)])
    plsc.subcore_barrier()

    @pl.loop(0, CNT_CHUNKS)
    def _(chunk):
        pltpu.sync_copy(ones_b, cacc.at[dst_v.at[chunk]], add=True)

    plsc.subcore_barrier()
    _tiled_copy(s, lambda o, z, r: cacc.at[pl.ds(o, r)],
                lambda o, z, r: cnt_ref.at[c, pl.ds(o, r)])


def _cnt_call(dst2d):
    mesh = plsc.VectorSubcoreMesh(core_axis_name="core",
                                  subcore_axis_name="subcore")
    fn = pl.kernel(
        _cnt_body,
        mesh=mesh,
        out_type=[jax.ShapeDtypeStruct((2, N, COLS), jnp.float32)],
        scratch_types=[
            pltpu.VMEM((CNT_CHUNKS, CNT_K), jnp.int32),
            pltpu.VMEM((CNT_K, COLS), jnp.float32),
            pltpu.VMEM_SHARED((N, COLS), jnp.float32),
        ],
    )
    z = jnp.zeros((ROWS_PER_TEC + 16, COLS), jnp.float32)
    return fn(dst2d, z)[0]


def _layer(x, pos, idxc, w1, b1, w2, b2, cnt_part, do_relu):
    fin = x.shape[1]
    hid = w1.shape[1]
    nb = hid // COLS
    a_t, q_t = _pre_call(x, pos, w1[:fin], w1[fin:], b1, nb)
    s_part = _edge_call(a_t, q_t, idxc, nb)
    return _post_call(s_part, cnt_part, w2, b2, nb, do_relu)


def kernel(h, pos, edge_index,
           c1_w1, c1_b1, c1_w2, c1_b2,
           c2_w1, c2_b1, c2_w2, c2_b2,
           c3_w1, c3_b1, c3_w2, c3_b2,
           c4_w1, c4_b1, c4_w2, c4_b2,
           m_w1, m_b1, m_w2, m_b2, m_w3, m_b3):
    ei = edge_index.astype(jnp.int32)
    idxc = jnp.stack([ei[0].reshape(E // K_EDGE, K_EDGE),
                      ei[1].reshape(E // K_EDGE, K_EDGE)], axis=1)
    dst2d_cnt = ei[1].reshape(E // CNT_K, CNT_K)

    cnt_part = _cnt_call(dst2d_cnt)
    x = _layer(h, pos, idxc, c1_w1, c1_b1, c1_w2, c1_b2, cnt_part, True)
    x = _layer(x, pos, idxc, c2_w1, c2_b1, c2_w2, c2_b2, cnt_part, True)
    x = _layer(x, pos, idxc, c3_w1, c3_b1, c3_w2, c3_b2, cnt_part, True)
    x = _layer(x, pos, idxc, c4_w1, c4_b1, c4_w2, c4_b2, cnt_part, True)
    return _mlp_call(x, m_w1, m_b1, m_w2, m_b2, m_w3, m_b3)
